# Initial kernel scaffold; baseline (speedup 1.0000x reference)
import jax
import jax.numpy as jnp
from jax.experimental import pallas as pl


def kernel(prior, emb):
    raise NotImplementedError("write your pallas kernel here")


# speedup vs baseline: 1.4010x; 1.4010x over previous
import functools

import jax
import jax.numpy as jnp
from jax import lax
from jax.experimental import pallas as pl
from jax.experimental.pallas import tpu as pltpu
from jax.experimental.pallas import tpu_sc as plsc

CHANNELS = 32
IDX_W = 128


@functools.lru_cache(maxsize=None)
def _build(n_rows: int, channels: int):
    info = plsc.get_sparse_core_info()
    nw = info.num_cores * info.num_subcores
    nc = info.num_cores

    assert n_rows % (nw * IDX_W) == 0
    idx_rows = n_rows // IDX_W
    rows_per_w = idx_rows // nw
    k = 8
    assert rows_per_w % k == 0
    n_iter = rows_per_w // k
    chunk = k * IDX_W

    mesh = plsc.VectorSubcoreMesh(core_axis_name="c", subcore_axis_name="s")

    @functools.partial(
        pl.kernel,
        out_type=jax.ShapeDtypeStruct((n_rows, channels), jnp.float32),
        mesh=mesh,
        scratch_types=[
            pltpu.VMEM((k, IDX_W), jnp.int32),
            pltpu.VMEM((chunk, channels), jnp.float32),
            pltpu.SemaphoreType.DMA,
        ],
        compiler_params=pltpu.CompilerParams(use_tc_tiling_on_sc=False),
    )
    def lookup(emb_hbm, idx_hbm, out_hbm, idx_v, rows_v, sem):
        wid = lax.axis_index("s") * nc + lax.axis_index("c")
        row0 = wid * rows_per_w

        @pl.loop(0, n_iter)
        def _(it):
            r = row0 + it * k
            pltpu.sync_copy(idx_hbm.at[pl.ds(r, k)], idx_v)
            copies = [
                pltpu.async_copy(
                    emb_hbm.at[idx_v.at[j]],
                    rows_v.at[pl.ds(j * IDX_W, IDX_W)],
                    sem,
                )
                for j in range(k)
            ]
            for cp in copies:
                cp.wait()
            pltpu.sync_copy(rows_v, out_hbm.at[pl.ds(r * IDX_W, chunk)])

    return lookup


def kernel(prior, emb):
    n_rows = prior.size
    idx = prior.reshape(n_rows // IDX_W, IDX_W)
    return _build(n_rows, emb.shape[1])(emb, idx)


# speedup vs baseline: 2.6059x; 1.8600x over previous
import functools

import jax
import jax.numpy as jnp
from jax import lax
from jax.experimental import pallas as pl
from jax.experimental.pallas import tpu as pltpu
from jax.experimental.pallas import tpu_sc as plsc

LANES = 16
CHUNK = 1024


@functools.lru_cache(maxsize=None)
def _build(n_rows: int, vocab: int, channels: int):
    info = plsc.get_sparse_core_info()
    nw = info.num_cores * info.num_subcores
    nc = info.num_cores

    assert n_rows % (nw * CHUNK) == 0
    rows_per_w = n_rows // nw
    n_iter = rows_per_w // CHUNK
    assert n_iter % 2 == 0
    groups = CHUNK // LANES
    cw = CHUNK * channels

    mesh = plsc.VectorSubcoreMesh(core_axis_name="c", subcore_axis_name="s")

    @functools.partial(
        pl.kernel,
        out_type=jax.ShapeDtypeStruct((n_rows * channels,), jnp.float32),
        mesh=mesh,
        scratch_types=[
            pltpu.VMEM((vocab * channels,), jnp.float32),
            pltpu.VMEM((2, CHUNK), jnp.int32),
            pltpu.VMEM((2, cw), jnp.float32),
            pltpu.SemaphoreType.DMA,
            pltpu.SemaphoreType.DMA,
        ],
        compiler_params=pltpu.CompilerParams(
            use_tc_tiling_on_sc=False, needs_layout_passes=False
        ),
    )
    def lookup(emb_hbm, idx_hbm, out_hbm, table_v, idx_v, buf_v, sem0, sem1):
        wid = lax.axis_index("s") * nc + lax.axis_index("c")
        row0 = wid * rows_per_w
        sems = (sem0, sem1)

        pltpu.sync_copy(emb_hbm, table_v)
        lane32 = lax.iota(jnp.int32, LANES) * channels

        @pl.loop(0, n_iter, step=2)
        def _(it):
            for b in range(2):
                i = it + b
                start = row0 + i * CHUNK
                bufb = buf_v.at[b]
                idxb = idx_v.at[b]
                pltpu.sync_copy(idx_hbm.at[pl.ds(start, CHUNK)], idxb)

                @pl.when(i >= 2)
                def _drain():
                    pltpu.make_async_copy(
                        out_hbm.at[pl.ds(start * channels, cw)], bufb, sems[b]
                    ).wait()

                @pl.loop(0, groups)
                def _(g):
                    idxv = idxb[pl.ds(g * LANES, LANES)] * channels
                    posv = lane32 + g * (LANES * channels)
                    for c in range(channels):
                        vals = plsc.load_gather(table_v, [idxv + c])
                        plsc.store_scatter(bufb, [posv + c], vals)

                pltpu.async_copy(
                    bufb, out_hbm.at[pl.ds(start * channels, cw)], sems[b]
                )

        for b in range(2):
            pltpu.make_async_copy(
                out_hbm.at[pl.ds(0, cw)], buf_v.at[b], sems[b]
            ).wait()

    return lookup


def kernel(prior, emb):
    n_rows = prior.size
    channels = emb.shape[1]
    idx = prior.reshape(n_rows)
    flat = _build(n_rows, emb.shape[0], channels)(emb.reshape(-1), idx)
    return flat.reshape(n_rows, channels)


# speedup vs baseline: 5.8563x; 2.2474x over previous
import functools

import jax
import jax.numpy as jnp
from jax import lax
from jax.experimental import pallas as pl
from jax.experimental.pallas import tpu as pltpu
from jax.experimental.pallas import tpu_sc as plsc

LANES = 16
CHUNK = 1024


@functools.lru_cache(maxsize=None)
def _build(n_rows: int, vocab: int, channels: int):
    info = plsc.get_sparse_core_info()
    nw = info.num_cores * info.num_subcores
    nc = info.num_cores

    assert n_rows % (nw * CHUNK) == 0
    rows_per_w = n_rows // nw
    n_iter = rows_per_w // CHUNK
    assert n_iter % 2 == 0
    groups = CHUNK // LANES
    cw = CHUNK * channels

    mesh = plsc.VectorSubcoreMesh(core_axis_name="c", subcore_axis_name="s")

    @functools.partial(
        pl.kernel,
        out_type=jax.ShapeDtypeStruct((n_rows * channels,), jnp.float32),
        mesh=mesh,
        scratch_types=[
            pltpu.VMEM((vocab * channels,), jnp.float32),
            pltpu.VMEM((2, CHUNK), jnp.int32),
            pltpu.VMEM((2, cw), jnp.float32),
            pltpu.SemaphoreType.DMA,
            pltpu.SemaphoreType.DMA,
            pltpu.SemaphoreType.DMA,
            pltpu.SemaphoreType.DMA,
        ],
        compiler_params=pltpu.CompilerParams(
            use_tc_tiling_on_sc=False, needs_layout_passes=False
        ),
    )
    def lookup(
        emb_hbm, idx_hbm, out_hbm, table_v, idx_v, buf_v, so0, so1, si0, si1
    ):
        wid = lax.axis_index("s") * nc + lax.axis_index("c")
        row0 = wid * rows_per_w
        out_sems = (so0, so1)
        idx_sems = (si0, si1)

        pltpu.sync_copy(emb_hbm, table_v)
        lane = lax.iota(jnp.int32, LANES)
        lane_ch = lane * channels

        pltpu.async_copy(
            idx_hbm.at[pl.ds(row0, CHUNK)], idx_v.at[0], idx_sems[0]
        )

        @pl.loop(0, n_iter, step=2)
        def _(it):
            for b in range(2):
                i = it + b
                start = row0 + i * CHUNK
                bufb = buf_v.at[b]
                idxb = idx_v.at[b]

                pltpu.make_async_copy(
                    idx_hbm.at[pl.ds(start, CHUNK)], idxb, idx_sems[b]
                ).wait()

                @pl.when(i + 1 < n_iter)
                def _prefetch():
                    pltpu.async_copy(
                        idx_hbm.at[pl.ds(start + CHUNK, CHUNK)],
                        idx_v.at[1 - b],
                        idx_sems[1 - b],
                    )

                @pl.when(i >= 2)
                def _drain():
                    pltpu.make_async_copy(
                        out_hbm.at[pl.ds(start * channels, cw)],
                        bufb,
                        out_sems[b],
                    ).wait()

                @pl.loop(0, groups)
                def _(g):
                    idxv = idxb[pl.ds(g * LANES, LANES)] * channels
                    posv = lane_ch + g * (LANES * channels)
                    for c in range(channels):
                        chv = (lane + c) & (channels - 1)
                        vals = plsc.load_gather(table_v, [idxv + chv])
                        plsc.store_scatter(bufb, [posv + chv], vals)

                pltpu.async_copy(
                    bufb, out_hbm.at[pl.ds(start * channels, cw)], out_sems[b]
                )

        for b in range(2):
            pltpu.make_async_copy(
                out_hbm.at[pl.ds(0, cw)], buf_v.at[b], out_sems[b]
            ).wait()

    return lookup


def kernel(prior, emb):
    n_rows = prior.size
    channels = emb.shape[1]
    idx = prior.reshape(n_rows)
    flat = _build(n_rows, emb.shape[0], channels)(emb.reshape(-1), idx)
    return flat.reshape(n_rows, channels)
